# baseline (speedup 1.0000x reference)
import functools
import math

import jax
import jax.numpy as jnp
from jax import lax
from jax.experimental import pallas as pl

N0 = 4096
E = 65536
H = 128
K0 = 2048
K1 = 1024


def _sq_body(nsteps, r_ref, ct_ref, a_ref, deg_ref):
    i, j, k = pl.program_id(0), pl.program_id(1), pl.program_id(2)

    @pl.when(k == 0)
    def _():
        a_ref[...] = jnp.zeros_like(a_ref)

    a_ref[...] += lax.dot_general(
        r_ref[...], ct_ref[...], (((1,), (1,)), ((), ())),
        preferred_element_type=jnp.float32)

    @pl.when(k == nsteps - 1)
    def _():
        blk = a_ref[...]
        bm, bn = blk.shape
        rows = lax.broadcasted_iota(jnp.int32, (bm, bn), 0) + i * bm
        cols = lax.broadcasted_iota(jnp.int32, (bm, bn), 1) + j * bn
        blk = jnp.where(rows == cols, 0.0, blk)
        a_ref[...] = blk
        rs = jnp.sum(blk, axis=1)

        @pl.when(j == 0)
        def _():
            deg_ref[...] = rs

        @pl.when(j != 0)
        def _():
            deg_ref[...] += rs


def _square_pool(Rm, CTm, bm=512, bn=512, bk=2048):
    m, K = Rm.shape
    nsteps = K // bk
    grid = (m // bm, m // bn, nsteps)
    return pl.pallas_call(
        functools.partial(_sq_body, nsteps),
        grid=grid,
        in_specs=[
            pl.BlockSpec((bm, bk), lambda i, j, k: (i, k)),
            pl.BlockSpec((bn, bk), lambda i, j, k: (j, k)),
        ],
        out_specs=[
            pl.BlockSpec((bm, bn), lambda i, j, k: (i, j)),
            pl.BlockSpec((bm,), lambda i, j, k: (i,)),
        ],
        out_shape=[
            jax.ShapeDtypeStruct((m, m), jnp.float32),
            jax.ShapeDtypeStruct((m,), jnp.float32),
        ],
    )(Rm, CTm)


def _conv_body(a_ref, z_ref, degk_ref, degi_ref, b_ref, y_ref):
    degk = degk_ref[...]
    dinvk = jnp.where(degk > 0, lax.rsqrt(jnp.maximum(degk, 1e-12)), 0.0)
    zs = z_ref[...] * dinvk[:, None]
    acc = jnp.dot(a_ref[...], zs, preferred_element_type=jnp.float32)
    degi = degi_ref[...]
    dinvi = jnp.where(degi > 0, lax.rsqrt(jnp.maximum(degi, 1e-12)), 0.0)
    y_ref[...] = jnp.maximum(acc * dinvi[:, None] + b_ref[...], 0.0)


def _conv_dense(A, Z, deg, b, bm=512):
    m = A.shape[0]
    return pl.pallas_call(
        _conv_body,
        grid=(m // bm,),
        in_specs=[
            pl.BlockSpec((bm, m), lambda i: (i, 0)),
            pl.BlockSpec((m, H), lambda i: (0, 0)),
            pl.BlockSpec((m,), lambda i: (0,)),
            pl.BlockSpec((bm,), lambda i: (i,)),
            pl.BlockSpec((1, H), lambda i: (0, 0)),
        ],
        out_specs=pl.BlockSpec((bm, H), lambda i: (i, 0)),
        out_shape=jax.ShapeDtypeStruct((m, H), jnp.float32),
    )(A, Z, deg, deg, b.reshape(1, H))


def _mm_body(x_ref, w_ref, s_ref, y_ref):
    y_ref[...] = jnp.dot(x_ref[...] * s_ref[...][:, None], w_ref[...],
                         preferred_element_type=jnp.float32)


def _mm(X, W, s=None, bm=1024):
    m, f = X.shape
    if s is None:
        s = jnp.ones((m,), jnp.float32)
    return pl.pallas_call(
        _mm_body,
        grid=(m // bm,),
        in_specs=[
            pl.BlockSpec((bm, f), lambda i: (i, 0)),
            pl.BlockSpec((f, W.shape[1]), lambda i: (0, 0)),
            pl.BlockSpec((bm,), lambda i: (i,)),
        ],
        out_specs=pl.BlockSpec((bm, W.shape[1]), lambda i: (i, 0)),
        out_shape=jax.ShapeDtypeStruct((m, W.shape[1]), jnp.float32),
    )(X, W, s)


def _topk_set(score, k):
    _, perm = lax.top_k(score, k)
    return jnp.sort(perm)


def kernel(x, edge_index, edge_weight, W0, b0, W1, b1, W2, b2, U0, c0, U1, c1, p0, p1):
    xf = x.reshape(N0, H)
    dst, src = edge_index[1], edge_index[0]

    deg0 = jnp.zeros((N0,), jnp.float32).at[dst].add(edge_weight)
    dinv0 = jnp.where(deg0 > 0, lax.rsqrt(jnp.maximum(deg0, 1e-12)), 0.0)
    wn = dinv0[dst] * edge_weight * dinv0[src]

    def spmm0(z):
        return jnp.zeros_like(z).at[dst].add(wn[:, None] * z[src])

    x0 = jax.nn.relu(spmm0(_mm(xf, W0)) + b0)

    score0 = _mm(x0, p0.reshape(H, 1)).reshape(N0) / jnp.linalg.norm(p0)
    perm0 = _topk_set(score0, K0)
    t0 = jnp.tanh(score0)[perm0]

    rank0 = jnp.full((N0,), -1, jnp.int32).at[perm0].set(
        jnp.arange(K0, dtype=jnp.int32))
    offdiag = dst != src
    rd = jnp.where(offdiag, rank0[dst], -1)
    rs = jnp.where(offdiag, rank0[src], -1)
    R0 = jnp.zeros((K0, N0), jnp.float32)
    R0 = R0.at[jnp.where(rd >= 0, rd, K0), src].add(edge_weight, mode="drop")
    R0 = R0.at[jnp.arange(K0), perm0].add(1.0)
    CT0 = jnp.zeros((K0, N0), jnp.float32)
    CT0 = CT0.at[jnp.where(rs >= 0, rs, K0), dst].add(edge_weight, mode="drop")
    CT0 = CT0.at[jnp.arange(K0), perm0].add(1.0)

    A1, deg1 = _square_pool(R0, CT0)

    x1 = _conv_dense(A1, _mm(x0[perm0], W1, s=t0), deg1, b1)

    score1 = _mm(x1, p1.reshape(H, 1), bm=512).reshape(K0) / jnp.linalg.norm(p1)
    perm1 = _topk_set(score1, K1)
    t1 = jnp.tanh(score1)[perm1]

    eyeadd = jnp.zeros((K1, K0), jnp.float32).at[
        jnp.arange(K1), perm1].set(1.0)
    R1 = A1[perm1, :] + eyeadd
    CT1 = A1.T[perm1, :] + eyeadd

    A2, deg2 = _square_pool(R1, CT1, bm=512, bn=512, bk=2048)

    x2 = _conv_dense(A2, _mm(x1[perm1], W2, s=t1, bm=512), deg2, b2)

    z1 = _mm(x1, U0[:H]) + jnp.zeros((K0, H), jnp.float32).at[perm1].set(
        _mm(x2, U0[H:]))
    x3 = _conv_dense(A1, z1, deg1, c0)

    z0 = _mm(x0, U1[:H]) + jnp.zeros((N0, H), jnp.float32).at[perm0].set(
        _mm(x3, U1[H:]))
    out = jax.nn.relu(spmm0(z0) + c1)
    return out.reshape(1, N0, H)


# speedup vs baseline: 1.9266x; 1.9266x over previous
import functools
import math

import jax
import jax.numpy as jnp
from jax import lax
from jax.experimental import pallas as pl
from jax.experimental.pallas import tpu as pltpu, tpu_sc as plsc

N0 = 4096
E = 65536
H = 128
K0 = 2048
K1 = 1024

_SC_MESH = plsc.VectorSubcoreMesh(core_axis_name="c", subcore_axis_name="s")
_NSC = 2
_NT = 16
_CH = 128
_EPT = E // (_NSC * _NT)


def _dinv_of(deg):
    return jnp.where(deg > 0, lax.rsqrt(jnp.maximum(deg, 1e-12)), 0.0)


@functools.partial(
    pl.kernel, mesh=_SC_MESH,
    out_type=jax.ShapeDtypeStruct((_NSC, N0), jnp.float32),
    scratch_types=[
        pltpu.VMEM((_CH,), jnp.int32),
        pltpu.VMEM((_CH,), jnp.float32),
        pltpu.VMEM((N0 // _NT,), jnp.float32),
        pltpu.VMEM_SHARED((N0,), jnp.float32),
    ],
)
def _sc_deg(dst_hbm, w_hbm, out_hbm, idx_v, w_v, zd_v, deg_sh):
    cid = lax.axis_index("c")
    sid = lax.axis_index("s")
    sl = N0 // _NT

    def zf(i, _):
        zd_v[pl.ds(i * 16, 16)] = jnp.zeros((16,), jnp.float32)
        return 0
    lax.fori_loop(0, sl // 16, zf, 0)
    pltpu.sync_copy(zd_v, deg_sh.at[pl.ds(sid * sl, sl)])
    plsc.subcore_barrier()

    base0 = cid * (E // _NSC) + sid * _EPT

    def chunk(ci, _):
        base = base0 + ci * _CH
        pltpu.sync_copy(dst_hbm.at[pl.ds(base, _CH)], idx_v)
        pltpu.sync_copy(w_hbm.at[pl.ds(base, _CH)], w_v)
        pltpu.sync_copy(w_v, deg_sh.at[idx_v], add=True)
        return 0
    lax.fori_loop(0, _EPT // _CH, chunk, 0)
    plsc.subcore_barrier()
    pltpu.sync_copy(deg_sh.at[pl.ds(sid * sl, sl)],
                    out_hbm.at[cid, pl.ds(sid * sl, sl)])


@functools.partial(
    pl.kernel, mesh=_SC_MESH,
    out_type=jax.ShapeDtypeStruct((_NSC, N0, H), jnp.float32),
    scratch_types=[
        pltpu.VMEM((_CH,), jnp.int32),
        pltpu.VMEM((_CH,), jnp.int32),
        pltpu.VMEM((_CH,), jnp.float32),
        pltpu.VMEM((_CH, H), jnp.float32),
        pltpu.VMEM((16, H), jnp.float32),
        pltpu.VMEM_SHARED((N0, H), jnp.float32),
        pltpu.SemaphoreType.DMA,
    ],
)
def _sc_spmm(z_hbm, src_hbm, dst_hbm, w_hbm, out_hbm,
             src_v, dst_v, w_v, rows_v, zb_v, agg_sh, sem):
    cid = lax.axis_index("c")
    sid = lax.axis_index("s")
    sl = N0 // _NT

    def zf(i, _):
        for f in range(H // 16):
            zb_v[i, pl.ds(f * 16, 16)] = jnp.zeros((16,), jnp.float32)
        return 0
    lax.fori_loop(0, 16, zf, 0)

    def zs(i, _):
        pltpu.sync_copy(zb_v, agg_sh.at[pl.ds(sid * sl + i * 16, 16)])
        return 0
    lax.fori_loop(0, sl // 16, zs, 0)
    plsc.subcore_barrier()

    base0 = cid * (E // _NSC) + sid * _EPT

    def chunk(ci, _):
        base = base0 + ci * _CH
        pltpu.sync_copy(src_hbm.at[pl.ds(base, _CH)], src_v)
        pltpu.sync_copy(dst_hbm.at[pl.ds(base, _CH)], dst_v)
        pltpu.sync_copy(w_hbm.at[pl.ds(base, _CH)], w_v)
        pltpu.async_copy(z_hbm.at[src_v], rows_v, sem).wait()

        def scale(g, _):
            wg = w_v[pl.ds(g * 16, 16)]
            for l in range(16):
                e = g * 16 + l
                wb = wg[l]
                for f in range(H // 16):
                    rows_v[e, pl.ds(f * 16, 16)] = (
                        rows_v[e, pl.ds(f * 16, 16)] * wb)
            return 0
        lax.fori_loop(0, _CH // 16, scale, 0)
        pltpu.sync_copy(rows_v, agg_sh.at[dst_v], add=True)
        return 0
    lax.fori_loop(0, _EPT // _CH, chunk, 0)
    plsc.subcore_barrier()
    pltpu.sync_copy(agg_sh.at[pl.ds(sid * sl, sl)],
                    out_hbm.at[cid, pl.ds(sid * sl, sl)])


def _fin_body(p_ref, deg_ref, b_ref, pv_ref, y_ref, s_ref):
    deg = deg_ref[0, :] + deg_ref[1, :]
    dinv = _dinv_of(deg)
    acc = p_ref[0] + p_ref[1]
    y = jnp.maximum(acc * dinv[:, None] + b_ref[...], 0.0)
    y_ref[...] = y
    pv = pv_ref[...]
    pn = pv / jnp.sqrt(jnp.sum(pv * pv))
    s_ref[...] = jnp.dot(y, pn.reshape(H, 1), preferred_element_type=jnp.float32)


def _fin(P, degP, b, pvec, bm=1024):
    m = P.shape[1]
    return pl.pallas_call(
        _fin_body,
        grid=(m // bm,),
        in_specs=[
            pl.BlockSpec((2, bm, H), lambda i: (0, i, 0)),
            pl.BlockSpec((2, bm), lambda i: (0, i)),
            pl.BlockSpec((1, H), lambda i: (0, 0)),
            pl.BlockSpec((1, H), lambda i: (0, 0)),
        ],
        out_specs=[
            pl.BlockSpec((bm, H), lambda i: (i, 0)),
            pl.BlockSpec((bm, 1), lambda i: (i, 0)),
        ],
        out_shape=[
            jax.ShapeDtypeStruct((m, H), jnp.float32),
            jax.ShapeDtypeStruct((m, 1), jnp.float32),
        ],
    )(P, degP, b.reshape(1, H), pvec.reshape(1, H))


def _sq_body(nsteps, r_ref, ct_ref, a_ref, deg_ref):
    i, j, k = pl.program_id(0), pl.program_id(1), pl.program_id(2)

    @pl.when(k == 0)
    def _():
        a_ref[...] = jnp.zeros_like(a_ref)

    a_ref[...] += lax.dot_general(
        r_ref[...], ct_ref[...], (((1,), (1,)), ((), ())),
        preferred_element_type=jnp.float32)

    @pl.when(k == nsteps - 1)
    def _():
        blk = a_ref[...]
        bm, bn = blk.shape
        rows = lax.broadcasted_iota(jnp.int32, (bm, bn), 0) + i * bm
        cols = lax.broadcasted_iota(jnp.int32, (bm, bn), 1) + j * bn
        blk = jnp.where(rows == cols, 0.0, blk)
        a_ref[...] = blk
        rs = jnp.sum(blk, axis=1)

        @pl.when(j == 0)
        def _():
            deg_ref[...] = rs

        @pl.when(j != 0)
        def _():
            deg_ref[...] += rs


def _square_pool(Rm, CTm, bm=512, bn=512, bk=2048):
    m, K = Rm.shape
    nsteps = K // bk
    grid = (m // bm, m // bn, nsteps)
    return pl.pallas_call(
        functools.partial(_sq_body, nsteps),
        grid=grid,
        in_specs=[
            pl.BlockSpec((bm, bk), lambda i, j, k: (i, k)),
            pl.BlockSpec((bn, bk), lambda i, j, k: (j, k)),
        ],
        out_specs=[
            pl.BlockSpec((bm, bn), lambda i, j, k: (i, j)),
            pl.BlockSpec((bm,), lambda i, j, k: (i,)),
        ],
        out_shape=[
            jax.ShapeDtypeStruct((m, m), jnp.float32),
            jax.ShapeDtypeStruct((m,), jnp.float32),
        ],
    )(Rm, CTm)


def _conv_body(a_ref, z1_ref, z2_ref, degk_ref, degi_ref, b_ref, pv_ref,
               y_ref, s_ref):
    dinvk = _dinv_of(degk_ref[...])
    zs = (z1_ref[...] + z2_ref[...]) * dinvk[:, None]
    acc = jnp.dot(a_ref[...], zs, preferred_element_type=jnp.float32)
    dinvi = _dinv_of(degi_ref[...])
    y = jnp.maximum(acc * dinvi[:, None] + b_ref[...], 0.0)
    y_ref[...] = y
    pv = pv_ref[...]
    pn = pv / jnp.sqrt(jnp.sum(pv * pv))
    s_ref[...] = jnp.dot(y, pn.reshape(H, 1), preferred_element_type=jnp.float32)


def _conv_dense(A, Z1, deg, b, pvec, Z2=None, bm=512):
    m = A.shape[0]
    if Z2 is None:
        Z2 = jnp.zeros_like(Z1)
    return pl.pallas_call(
        _conv_body,
        grid=(m // bm,),
        in_specs=[
            pl.BlockSpec((bm, m), lambda i: (i, 0)),
            pl.BlockSpec((m, H), lambda i: (0, 0)),
            pl.BlockSpec((m, H), lambda i: (0, 0)),
            pl.BlockSpec((m,), lambda i: (0,)),
            pl.BlockSpec((bm,), lambda i: (i,)),
            pl.BlockSpec((1, H), lambda i: (0, 0)),
            pl.BlockSpec((1, H), lambda i: (0, 0)),
        ],
        out_specs=[
            pl.BlockSpec((bm, H), lambda i: (i, 0)),
            pl.BlockSpec((bm, 1), lambda i: (i, 0)),
        ],
        out_shape=[
            jax.ShapeDtypeStruct((m, H), jnp.float32),
            jax.ShapeDtypeStruct((m, 1), jnp.float32),
        ],
    )(A, Z1, Z2, deg, deg, b.reshape(1, H), pvec.reshape(1, H))


def _mm_body(x_ref, w_ref, si_ref, so_ref, y_ref):
    y = jnp.dot(x_ref[...] * si_ref[...][:, None], w_ref[...],
                preferred_element_type=jnp.float32)
    y_ref[...] = y * so_ref[...][:, None]


def _mm(X, W, s_in=None, s_out=None, bm=1024):
    m, f = X.shape
    if s_in is None:
        s_in = jnp.ones((m,), jnp.float32)
    if s_out is None:
        s_out = jnp.ones((m,), jnp.float32)
    return pl.pallas_call(
        _mm_body,
        grid=(m // bm,),
        in_specs=[
            pl.BlockSpec((bm, f), lambda i: (i, 0)),
            pl.BlockSpec((f, W.shape[1]), lambda i: (0, 0)),
            pl.BlockSpec((bm,), lambda i: (i,)),
            pl.BlockSpec((bm,), lambda i: (i,)),
        ],
        out_specs=pl.BlockSpec((bm, W.shape[1]), lambda i: (i, 0)),
        out_shape=jax.ShapeDtypeStruct((m, W.shape[1]), jnp.float32),
    )(X, W, s_in, s_out)


def _topk_set(score, k):
    _, perm = lax.top_k(score, k)
    return jnp.sort(perm)


def kernel(x, edge_index, edge_weight, W0, b0, W1, b1, W2, b2, U0, c0, U1, c1, p0, p1):
    xf = x.reshape(N0, H)
    dst, src = edge_index[1], edge_index[0]

    degP = _sc_deg(dst, edge_weight)
    deg0 = degP[0] + degP[1]
    dinv0 = _dinv_of(deg0)

    z0p = _mm(xf, W0, s_out=dinv0)
    P = _sc_spmm(z0p, src, dst, edge_weight)
    x0, s0 = _fin(P, degP, b0, p0)
    score0 = s0.reshape(N0)

    perm0 = _topk_set(score0, K0)
    t0 = jnp.tanh(score0)[perm0]

    rank0 = jnp.full((N0,), -1, jnp.int32).at[perm0].set(
        jnp.arange(K0, dtype=jnp.int32))
    offdiag = dst != src
    rd = jnp.where(offdiag, rank0[dst], -1)
    rs = jnp.where(offdiag, rank0[src], -1)
    R0 = jnp.zeros((K0, N0), jnp.float32)
    R0 = R0.at[jnp.where(rd >= 0, rd, K0), src].add(edge_weight, mode="drop")
    R0 = R0.at[jnp.arange(K0), perm0].add(1.0)
    CT0 = jnp.zeros((K0, N0), jnp.float32)
    CT0 = CT0.at[jnp.where(rs >= 0, rs, K0), dst].add(edge_weight, mode="drop")
    CT0 = CT0.at[jnp.arange(K0), perm0].add(1.0)

    A1, deg1 = _square_pool(R0, CT0)

    x1, s1 = _conv_dense(A1, _mm(x0[perm0], W1, s_in=t0), deg1, b1, p1)
    score1 = s1.reshape(K0)

    perm1 = _topk_set(score1, K1)
    t1 = jnp.tanh(score1)[perm1]

    eyeadd = jnp.zeros((K1, K0), jnp.float32).at[
        jnp.arange(K1), perm1].set(1.0)
    R1 = A1[perm1, :] + eyeadd
    CT1 = A1.T[perm1, :] + eyeadd

    A2, deg2 = _square_pool(R1, CT1, bm=512, bn=512, bk=2048)

    x2, _ = _conv_dense(A2, _mm(x1[perm1], W2, s_in=t1, bm=512), deg2, b2, p1)

    up1 = jnp.zeros((K0, H), jnp.float32).at[perm1].set(_mm(x2, U0[H:]))
    x3, _ = _conv_dense(A1, _mm(x1, U0[:H]), deg1, c0, p1, Z2=up1)

    u0 = _mm(x3, U1[H:], s_out=dinv0[perm0])
    zup = _mm(x0, U1[:H], s_out=dinv0) + jnp.zeros(
        (N0, H), jnp.float32).at[perm0].set(u0)
    Q = _sc_spmm(zup, src, dst, edge_weight)
    out, _ = _fin(Q, degP, c1, p1)
    return out.reshape(1, N0, H)
